# Initial kernel scaffold; baseline (speedup 1.0000x reference)
import functools

import jax
import jax.numpy as jnp
from jax import lax
from jax.experimental import pallas as pl
from jax.experimental.pallas import tpu as pltpu
from jax.experimental.pallas import tpu_sc as plsc

VOCAB = 100000
D = 64
B = 4096
L = 50

_NC = 2
_NS = 16
_NW = _NC * _NS
_QPT = B // _NW
_CHUNK_Q = 16
_NCHUNK = _QPT // _CHUNK_Q
_ROWS = _CHUNK_Q * L
_DMA_N = 80
_NDMA = _ROWS // _DMA_N


def _encoder_body(qflat_hbm, table_hbm, w_hbm, out_hbm,
                  idx_s, rows_s, w_s, e_s, out_s, sem):
    wid = lax.axis_index("s") * _NC + lax.axis_index("c")

    iota16 = lax.iota(jnp.int32, 16)
    q50 = iota16 * L
    zeros16 = jnp.zeros((16,), jnp.int32)

    for j in range(_NCHUNK):
        cg = wid * _NCHUNK + j
        base = cg * _ROWS

        pltpu.sync_copy(qflat_hbm.at[pl.ds(base, _ROWS)], idx_s)

        copies = []
        for k in range(_NDMA):
            off = _DMA_N * k
            isl = idx_s.at[pl.ds(off, _DMA_N)]
            copies.append(pltpu.async_copy(
                table_hbm.at[isl], rows_s.at[pl.ds(off, _DMA_N), :], sem))
            copies.append(pltpu.async_copy(
                w_hbm.at[isl], w_s.at[pl.ds(off, _DMA_N), :], sem))
        for c in copies:
            c.wait()

        def max_body(t, m):
            wt = plsc.load_gather(w_s, [q50 + t, zeros16])
            return jnp.maximum(m, wt)
        m = lax.fori_loop(0, L, max_body,
                          jnp.full((16,), -jnp.inf, jnp.float32))

        def exp_body(t, s):
            wt = plsc.load_gather(w_s, [q50 + t, zeros16])
            e = jnp.exp(wt - m)
            e_s[t] = e
            return s + e
        S = lax.fori_loop(0, L, exp_body, jnp.zeros((16,), jnp.float32))

        nsq = jnp.zeros((16,), jnp.float32)
        for dg in range(D // 16):
            def acc_body(t, accs):
                e = e_s[t]
                rowi = q50 + t
                out = []
                for dd in range(16):
                    col = jnp.full((16,), dg * 16 + dd, jnp.int32)
                    r = plsc.load_gather(rows_s, [rowi, col])
                    out.append(accs[dd] + e * r)
                return tuple(out)
            accs = lax.fori_loop(
                0, L, acc_body,
                tuple(jnp.zeros((16,), jnp.float32) for _ in range(16)))
            for dd in range(16):
                col = jnp.full((16,), dg * 16 + dd, jnp.int32)
                plsc.store_scatter(out_s, [iota16, col], accs[dd])
                nsq = nsq + accs[dd] * accs[dd]

        inv = 1.0 / (jnp.sqrt(nsq) + 1e-4 * S)

        def scale_body(d, carry):
            col = zeros16 + d
            v = plsc.load_gather(out_s, [iota16, col])
            plsc.store_scatter(out_s, [iota16, col], v * inv)
            return carry
        lax.fori_loop(0, D, scale_body, 0)

        pltpu.sync_copy(out_s, out_hbm.at[pl.ds(cg * _CHUNK_Q, 16), :])


@jax.jit
def _encode(qflat, table, w_table):
    mesh = plsc.VectorSubcoreMesh(core_axis_name="c", subcore_axis_name="s")
    return pl.kernel(
        _encoder_body,
        mesh=mesh,
        out_type=jax.ShapeDtypeStruct((B, D), jnp.float32),
        scratch_types=[
            pltpu.VMEM((_ROWS,), jnp.int32),
            pltpu.VMEM((_ROWS, D), jnp.float32),
            pltpu.VMEM((_ROWS, 1), jnp.float32),
            pltpu.VMEM((L, 16), jnp.float32),
            pltpu.VMEM((_CHUNK_Q, D), jnp.float32),
            pltpu.SemaphoreType.DMA,
        ],
    )(qflat, table, w_table)


def kernel(query, table, w_table):
    return _encode(query.reshape(-1), table, w_table)


# speedup vs baseline: 8.4853x; 8.4853x over previous
import functools

import jax
import jax.numpy as jnp
from jax import lax
from jax.experimental import pallas as pl
from jax.experimental.pallas import tpu as pltpu
from jax.experimental.pallas import tpu_sc as plsc

VOCAB = 100000
D = 64
B = 4096
L = 50

_NC = 2
_NS = 16
_NW = _NC * _NS
_QPT = B // _NW
_CHUNK_Q = 16
_NCHUNK = _QPT // _CHUNK_Q
_ROWS = _CHUNK_Q * L
_DMA_N = 80
_NDMA = _ROWS // _DMA_N


def _encoder_body(qflat_hbm, table_hbm, w_hbm, out_hbm,
                  idx_s, rows_s, w_s, e_s, out_s, sem):
    wid = lax.axis_index("s") * _NC + lax.axis_index("c")

    iota16 = lax.iota(jnp.int32, 16)
    q50 = iota16 * L
    zeros16 = jnp.zeros((16,), jnp.int32)

    for j in range(_NCHUNK):
        cg = wid * _NCHUNK + j
        base = cg * _ROWS

        pltpu.sync_copy(qflat_hbm.at[pl.ds(base, _ROWS)], idx_s)

        copies = []
        for k in range(_NDMA):
            off = _DMA_N * k
            isl = idx_s.at[pl.ds(off, _DMA_N)]
            copies.append(pltpu.async_copy(
                table_hbm.at[isl], rows_s.at[pl.ds(off, _DMA_N), :], sem))
            copies.append(pltpu.async_copy(
                w_hbm.at[isl], w_s.at[pl.ds(off, _DMA_N)], sem))
        for c in copies:
            c.wait()

        def max_body(t, m):
            wt = plsc.load_gather(w_s, [q50 + t])
            return jnp.maximum(m, wt)
        m = lax.fori_loop(0, L, max_body,
                          jnp.full((16,), -jnp.inf, jnp.float32))

        def exp_body(t, s):
            wt = plsc.load_gather(w_s, [q50 + t])
            e = jnp.exp(wt - m)
            e_s[t] = e
            return s + e
        S = lax.fori_loop(0, L, exp_body, jnp.zeros((16,), jnp.float32))

        nsq = jnp.zeros((16,), jnp.float32)
        for dg in range(D // 16):
            def acc_body(t, accs):
                e = e_s[t]
                rowi = q50 + t
                out = []
                for dd in range(16):
                    col = jnp.full((16,), dg * 16 + dd, jnp.int32)
                    r = plsc.load_gather(rows_s, [rowi, col])
                    out.append(accs[dd] + e * r)
                return tuple(out)
            accs = lax.fori_loop(
                0, L, acc_body,
                tuple(jnp.zeros((16,), jnp.float32) for _ in range(16)))
            for dd in range(16):
                col = jnp.full((16,), dg * 16 + dd, jnp.int32)
                plsc.store_scatter(out_s, [iota16, col], accs[dd])
                nsq = nsq + accs[dd] * accs[dd]

        i = plsc.bitcast(nsq, jnp.int32)
        y = plsc.bitcast(jnp.int32(0x5F3759DF) - (i >> 1), jnp.float32)
        hx = nsq * 0.5
        for _ in range(3):
            y = y * (1.5 - hx * y * y)
        norm = nsq * y
        inv = 1.0 / (norm + 1e-4 * S)

        def scale_body(d, carry):
            col = zeros16 + d
            v = plsc.load_gather(out_s, [iota16, col])
            plsc.store_scatter(out_s, [iota16, col], v * inv)
            return carry
        lax.fori_loop(0, D, scale_body, 0)

        pltpu.sync_copy(out_s, out_hbm.at[pl.ds(cg * _CHUNK_Q, 16), :])


@jax.jit
def _encode(qflat, table, w_table):
    mesh = plsc.VectorSubcoreMesh(core_axis_name="c", subcore_axis_name="s")
    return pl.kernel(
        _encoder_body,
        mesh=mesh,
        out_type=jax.ShapeDtypeStruct((B, D), jnp.float32),
        compiler_params=pltpu.CompilerParams(
            needs_layout_passes=False, use_tc_tiling_on_sc=False),
        scratch_types=[
            pltpu.VMEM((_ROWS,), jnp.int32),
            pltpu.VMEM((_ROWS, D), jnp.float32),
            pltpu.VMEM((_ROWS,), jnp.float32),
            pltpu.VMEM((L, 16), jnp.float32),
            pltpu.VMEM((_CHUNK_Q, D), jnp.float32),
            pltpu.SemaphoreType.DMA,
        ],
    )(qflat, table, w_table)


def kernel(query, table, w_table):
    return _encode(query.reshape(-1), table, w_table.reshape(-1))


# speedup vs baseline: 16.4259x; 1.9358x over previous
import functools

import jax
import jax.numpy as jnp
from jax import lax
from jax.experimental import pallas as pl
from jax.experimental.pallas import tpu as pltpu
from jax.experimental.pallas import tpu_sc as plsc

VOCAB = 100000
D = 64
B = 4096
L = 50

_NC = 2
_NS = 16
_NW = _NC * _NS
_QPT = B // _NW
_CHUNK_Q = 16
_NCHUNK = _QPT // _CHUNK_Q
_ROWS = _CHUNK_Q * L
_DMA_N = 80
_NDMA = _ROWS // _DMA_N


def _encoder_body(qflat_hbm, table_hbm, w_hbm, out_hbm,
                  idx_s, rows_s, w_s, e_s, out_s, sem):
    wid = lax.axis_index("s") * _NC + lax.axis_index("c")

    iota16 = lax.iota(jnp.int32, 16)
    q50 = iota16 * L
    zeros16 = jnp.zeros((16,), jnp.int32)

    for j in range(_NCHUNK):
        cg = wid * _NCHUNK + j
        base = cg * _ROWS

        pltpu.sync_copy(qflat_hbm.at[pl.ds(base, _ROWS)], idx_s)

        copies = []
        for k in range(_NDMA):
            off = _DMA_N * k
            isl = idx_s.at[pl.ds(off, _DMA_N)]
            copies.append(pltpu.async_copy(
                table_hbm.at[isl], rows_s.at[pl.ds(off, _DMA_N), :], sem))
            copies.append(pltpu.async_copy(
                w_hbm.at[isl], w_s.at[pl.ds(off, _DMA_N)], sem))
        for c in copies:
            c.wait()

        def max_body(t, m):
            wt = plsc.load_gather(w_s, [q50 + t])
            return jnp.maximum(m, wt)
        m = lax.fori_loop(0, L, max_body,
                          jnp.full((16,), -jnp.inf, jnp.float32))

        def exp_body(t, s):
            wt = plsc.load_gather(w_s, [q50 + t])
            e = jnp.exp(wt - m)
            e_s[t] = e
            return s + e
        S = lax.fori_loop(0, L, exp_body, jnp.zeros((16,), jnp.float32))

        rot = [(iota16 + dd) & 15 for dd in range(16)]
        nsq = jnp.zeros((16,), jnp.float32)
        for dg in range(D // 16):
            def acc_body(t, accs):
                e = e_s[t]
                rowi = q50 + t
                out = []
                for dd in range(16):
                    r = plsc.load_gather(rows_s, [rowi, rot[dd] + dg * 16])
                    out.append(accs[dd] + e * r)
                return tuple(out)
            accs = lax.fori_loop(
                0, L, acc_body,
                tuple(jnp.zeros((16,), jnp.float32) for _ in range(16)))
            for dd in range(16):
                plsc.store_scatter(out_s, [iota16, rot[dd] + dg * 16],
                                   accs[dd])
                nsq = nsq + accs[dd] * accs[dd]

        i = plsc.bitcast(nsq, jnp.int32)
        y = plsc.bitcast(jnp.int32(0x5F3759DF) - (i >> 1), jnp.float32)
        hx = nsq * 0.5
        for _ in range(3):
            y = y * (1.5 - hx * y * y)
        norm = nsq * y
        inv = 1.0 / (norm + 1e-4 * S)

        def scale_body(d, carry):
            col = (iota16 + d) & (D - 1)
            v = plsc.load_gather(out_s, [iota16, col])
            plsc.store_scatter(out_s, [iota16, col], v * inv)
            return carry
        lax.fori_loop(0, D, scale_body, 0)

        pltpu.sync_copy(out_s, out_hbm.at[pl.ds(cg * _CHUNK_Q, 16), :])


@jax.jit
def _encode(qflat, table, w_table):
    mesh = plsc.VectorSubcoreMesh(core_axis_name="c", subcore_axis_name="s")
    return pl.kernel(
        _encoder_body,
        mesh=mesh,
        out_type=jax.ShapeDtypeStruct((B, D), jnp.float32),
        compiler_params=pltpu.CompilerParams(
            needs_layout_passes=False, use_tc_tiling_on_sc=False),
        scratch_types=[
            pltpu.VMEM((_ROWS,), jnp.int32),
            pltpu.VMEM((_ROWS, D), jnp.float32),
            pltpu.VMEM((_ROWS,), jnp.float32),
            pltpu.VMEM((L, 16), jnp.float32),
            pltpu.VMEM((_CHUNK_Q, D), jnp.float32),
            pltpu.SemaphoreType.DMA,
        ],
    )(qflat, table, w_table)


def kernel(query, table, w_table):
    return _encode(query.reshape(-1), table, w_table.reshape(-1))


# speedup vs baseline: 18.7759x; 1.1431x over previous
import functools

import jax
import jax.numpy as jnp
from jax import lax
from jax.experimental import pallas as pl
from jax.experimental.pallas import tpu as pltpu
from jax.experimental.pallas import tpu_sc as plsc

VOCAB = 100000
D = 64
B = 4096
L = 50

_NC = 2
_NS = 16
_NW = _NC * _NS
_QPT = B // _NW
_CHUNK_Q = 16
_NCHUNK = _QPT // _CHUNK_Q
_ROWS = _CHUNK_Q * L
_DMA_N = 80
_NDMA = _ROWS // _DMA_N


def _encoder_body(qflat_hbm, table_hbm, w_hbm, out_hbm,
                  idx_s, rows_s, w_s, e_s, out_s, sem0, sem1):
    wid = lax.axis_index("s") * _NC + lax.axis_index("c")
    sems = (sem0, sem1)

    iota16 = lax.iota(jnp.int32, 16)
    q50 = iota16 * L
    zeros16 = jnp.zeros((16,), jnp.int32)

    def stage(j, b):
        cg = wid * _NCHUNK + j
        pltpu.sync_copy(qflat_hbm.at[pl.ds(cg * _ROWS, _ROWS)],
                        idx_s.at[b])
        copies = []
        for k in range(_NDMA):
            off = _DMA_N * k
            isl = idx_s.at[b].at[pl.ds(off, _DMA_N)]
            copies.append(pltpu.async_copy(
                table_hbm.at[isl], rows_s.at[b].at[pl.ds(off, _DMA_N), :],
                sems[b]))
            copies.append(pltpu.async_copy(
                w_hbm.at[isl], w_s.at[b].at[pl.ds(off, _DMA_N)], sems[b]))
        return copies

    pending = stage(0, 0)
    for j in range(_NCHUNK):
        b = j & 1
        cg = wid * _NCHUNK + j
        drain = pending
        if j + 1 < _NCHUNK:
            pending = stage(j + 1, 1 - b)
        for c in drain:
            c.wait()
        rows_b = rows_s.at[b]
        w_b = w_s.at[b]

        def max_body(t, m):
            wt = plsc.load_gather(w_b, [q50 + t])
            return jnp.maximum(m, wt)
        m = lax.fori_loop(0, L, max_body,
                          jnp.full((16,), -jnp.inf, jnp.float32))

        def exp_body(t, s):
            wt = plsc.load_gather(w_b, [q50 + t])
            e = jnp.exp(wt - m)
            e_s[t] = e
            return s + e
        S = lax.fori_loop(0, L, exp_body, jnp.zeros((16,), jnp.float32))

        rot = [(iota16 + dd) & 15 for dd in range(16)]
        nsq = jnp.zeros((16,), jnp.float32)
        for dg in range(D // 16):
            def acc_body(t, accs):
                e = e_s[t]
                rowi = q50 + t
                out = []
                for dd in range(16):
                    r = plsc.load_gather(rows_b, [rowi, rot[dd] + dg * 16])
                    out.append(accs[dd] + e * r)
                return tuple(out)
            accs = lax.fori_loop(
                0, L, acc_body,
                tuple(jnp.zeros((16,), jnp.float32) for _ in range(16)))
            for dd in range(16):
                plsc.store_scatter(out_s, [iota16, rot[dd] + dg * 16],
                                   accs[dd])
                nsq = nsq + accs[dd] * accs[dd]

        i = plsc.bitcast(nsq, jnp.int32)
        y = plsc.bitcast(jnp.int32(0x5F3759DF) - (i >> 1), jnp.float32)
        hx = nsq * 0.5
        for _ in range(3):
            y = y * (1.5 - hx * y * y)
        norm = nsq * y
        inv = 1.0 / (norm + 1e-4 * S)

        def scale_body(d, carry):
            col = (iota16 + d) & (D - 1)
            v = plsc.load_gather(out_s, [iota16, col])
            plsc.store_scatter(out_s, [iota16, col], v * inv)
            return carry
        lax.fori_loop(0, D, scale_body, 0)

        pltpu.sync_copy(out_s, out_hbm.at[pl.ds(cg * _CHUNK_Q, 16), :])


@jax.jit
def _encode(qflat, table, w_table):
    mesh = plsc.VectorSubcoreMesh(core_axis_name="c", subcore_axis_name="s")
    return pl.kernel(
        _encoder_body,
        mesh=mesh,
        out_type=jax.ShapeDtypeStruct((B, D), jnp.float32),
        compiler_params=pltpu.CompilerParams(
            needs_layout_passes=False, use_tc_tiling_on_sc=False),
        scratch_types=[
            pltpu.VMEM((2, _ROWS), jnp.int32),
            pltpu.VMEM((2, _ROWS, D), jnp.float32),
            pltpu.VMEM((2, _ROWS), jnp.float32),
            pltpu.VMEM((L, 16), jnp.float32),
            pltpu.VMEM((_CHUNK_Q, D), jnp.float32),
            pltpu.SemaphoreType.DMA,
            pltpu.SemaphoreType.DMA,
        ],
    )(qflat, table, w_table)


def kernel(query, table, w_table):
    return _encode(query.reshape(-1), table, w_table.reshape(-1))
